# Initial kernel scaffold; baseline (speedup 1.0000x reference)
import jax
import jax.numpy as jnp
from jax.experimental import pallas as pl


def kernel(node_feat, edge_index, W):
    raise NotImplementedError("write your pallas kernel here")


# speedup vs baseline: 5.1731x; 5.1731x over previous
import functools

import jax
import jax.numpy as jnp
from jax import lax
from jax.experimental import pallas as pl
from jax.experimental.pallas import tpu as pltpu
from jax.experimental.pallas import tpu_sc as plsc

N_NODES = 10000
N_EDGES = 320000
D = 128

NW = 32
BPW = N_EDGES // NW
C = 400
NCHUNK = BPW // C


def _mm_body(x_ref, w1_ref, w2_ref, p_ref, q_ref):
    x = x_ref[...]
    p_ref[...] = jnp.dot(x, w1_ref[...], preferred_element_type=jnp.float32)
    q_ref[...] = jnp.dot(x, w2_ref[...], preferred_element_type=jnp.float32)


def _project_nodes(node_feat, w1, w2):
    blk = 1000
    grid = N_NODES // blk
    return pl.pallas_call(
        _mm_body,
        grid=(grid,),
        in_specs=[
            pl.BlockSpec((blk, D), lambda i: (i, 0)),
            pl.BlockSpec((D, D), lambda i: (0, 0)),
            pl.BlockSpec((D, D), lambda i: (0, 0)),
        ],
        out_specs=[
            pl.BlockSpec((blk, D), lambda i: (i, 0)),
            pl.BlockSpec((blk, D), lambda i: (i, 0)),
        ],
        out_shape=[
            jax.ShapeDtypeStruct((N_NODES, D), jnp.float32),
            jax.ShapeDtypeStruct((N_NODES, D), jnp.float32),
        ],
    )(node_feat, w1, w2)


def _sc_body(p_hbm, q_hbm, src_hbm, dst_hbm, out_hbm,
             idxs, idxd, bufp, bufq, semp, semq):
    wid = lax.axis_index("s") * 2 + lax.axis_index("c")
    base = wid * BPW

    def chunk(t, carry):
        off = base + t * C
        pltpu.sync_copy(src_hbm.at[pl.ds(off, C)], idxs)
        pltpu.sync_copy(dst_hbm.at[pl.ds(off, C)], idxd)
        cp = pltpu.async_copy(p_hbm.at[idxs], bufp, semp)
        cq = pltpu.async_copy(q_hbm.at[idxd], bufq, semq)
        cp.wait()
        cq.wait()

        def row(r, carry2):
            for j in range(8):
                sl = pl.ds(j * 16, 16)
                s = bufp[r, sl] + bufq[r, sl]
                bufp[r, sl] = jnp.maximum(s, s * 0.01)
            return carry2

        lax.fori_loop(0, C, row, 0)
        pltpu.sync_copy(bufp, out_hbm.at[pl.ds(off, C)])
        return carry

    lax.fori_loop(0, NCHUNK, chunk, 0)


_sc_edges = functools.partial(
    pl.kernel,
    out_type=jax.ShapeDtypeStruct((N_EDGES, D), jnp.float32),
    mesh=plsc.VectorSubcoreMesh(core_axis_name="c", subcore_axis_name="s"),
    scratch_types=[
        pltpu.VMEM((C,), jnp.int32),
        pltpu.VMEM((C,), jnp.int32),
        pltpu.VMEM((C, D), jnp.float32),
        pltpu.VMEM((C, D), jnp.float32),
        pltpu.SemaphoreType.DMA,
        pltpu.SemaphoreType.DMA,
    ],
)(_sc_body)


def kernel(node_feat, edge_index, W):
    edge_index = edge_index.astype(jnp.int32)
    src = edge_index[0]
    dst = edge_index[1]
    w1 = W[:D]
    w2 = W[D:]
    P, Q = _project_nodes(node_feat, w1, w2)
    return _sc_edges(P, Q, src, dst)


# speedup vs baseline: 6.7567x; 1.3061x over previous
import functools

import jax
import jax.numpy as jnp
from jax import lax
from jax.experimental import pallas as pl
from jax.experimental.pallas import tpu as pltpu
from jax.experimental.pallas import tpu_sc as plsc

N_NODES = 10000
N_EDGES = 320000
D = 128

NW = 32
BPW = N_EDGES // NW
C = 200
NCHUNK = BPW // C


def _mm_body(x_ref, w1_ref, w2_ref, p_ref, q_ref):
    x = x_ref[...]
    p_ref[...] = jnp.dot(x, w1_ref[...], preferred_element_type=jnp.float32)
    q_ref[...] = jnp.dot(x, w2_ref[...], preferred_element_type=jnp.float32)


def _project_nodes(node_feat, w1, w2):
    blk = 1000
    grid = N_NODES // blk
    return pl.pallas_call(
        _mm_body,
        grid=(grid,),
        in_specs=[
            pl.BlockSpec((blk, D), lambda i: (i, 0)),
            pl.BlockSpec((D, D), lambda i: (0, 0)),
            pl.BlockSpec((D, D), lambda i: (0, 0)),
        ],
        out_specs=[
            pl.BlockSpec((blk, D), lambda i: (i, 0)),
            pl.BlockSpec((blk, D), lambda i: (i, 0)),
        ],
        out_shape=[
            jax.ShapeDtypeStruct((N_NODES, D), jnp.float32),
            jax.ShapeDtypeStruct((N_NODES, D), jnp.float32),
        ],
    )(node_feat, w1, w2)


def _sc_body(p_hbm, q_hbm, src_hbm, dst_hbm, out_hbm,
             idxs, idxd, buf0, buf1, sp0, sq0, sp1, sq1, so0, so1):
    wid = lax.axis_index("s") * 2 + lax.axis_index("c")
    base = wid * BPW

    pltpu.sync_copy(src_hbm.at[pl.ds(base, BPW)], idxs)
    pltpu.sync_copy(dst_hbm.at[pl.ds(base, BPW)], idxd)

    bufs = ((buf0, sp0, sq0, so0), (buf1, sp1, sq1, so1))

    def start_p(t, b):
        buf, sp, _, _ = bufs[b]
        return pltpu.async_copy(p_hbm.at[idxs.at[pl.ds(t * C, C)]], buf, sp)

    def start_q(t, b):
        buf, _, sq, _ = bufs[b]
        return pltpu.async_copy(q_hbm.at[idxd.at[pl.ds(t * C, C)]], buf, sq,
                                add=True)

    def wait_p(b):
        buf, sp, _, _ = bufs[b]
        pltpu.make_async_copy(p_hbm.at[idxs.at[pl.ds(0, C)]], buf, sp).wait()

    def wait_q(b):
        buf, _, sq, _ = bufs[b]
        pltpu.make_async_copy(q_hbm.at[idxd.at[pl.ds(0, C)]], buf, sq).wait()

    def start_out(t, b):
        buf, _, _, so = bufs[b]
        return pltpu.async_copy(buf, out_hbm.at[pl.ds(base + t * C, C)], so)

    def wait_out(b):
        buf, _, _, so = bufs[b]
        pltpu.make_async_copy(buf, out_hbm.at[pl.ds(base, C)], so).wait()

    start_p(0, 0)
    wait_p(0)
    start_q(0, 0)

    def half_iter(t, b):
        wait_q(b)

        @pl.when(t + 1 < NCHUNK)
        def _():
            @pl.when(t >= 1)
            def _():
                wait_out(1 - b)
            start_p(t + 1, 1 - b)

        @plsc.parallel_loop(0, C, step=1, unroll=2)
        def _(r):
            for j in range(8):
                sl = pl.ds(j * 16, 16)
                v = buf0[r, sl] if b == 0 else buf1[r, sl]
                o = jnp.maximum(v, v * 0.01)
                if b == 0:
                    buf0[r, sl] = o
                else:
                    buf1[r, sl] = o

        @pl.when(t + 1 < NCHUNK)
        def _():
            wait_p(1 - b)
            start_q(t + 1, 1 - b)

        start_out(t, b)

    def pair(g, carry):
        half_iter(2 * g, 0)
        half_iter(2 * g + 1, 1)
        return carry

    lax.fori_loop(0, NCHUNK // 2, pair, 0)
    wait_out(1)


_sc_edges = functools.partial(
    pl.kernel,
    out_type=jax.ShapeDtypeStruct((N_EDGES, D), jnp.float32),
    mesh=plsc.VectorSubcoreMesh(core_axis_name="c", subcore_axis_name="s"),
    scratch_types=[
        pltpu.VMEM((BPW,), jnp.int32),
        pltpu.VMEM((BPW,), jnp.int32),
        pltpu.VMEM((C, D), jnp.float32),
        pltpu.VMEM((C, D), jnp.float32),
        pltpu.SemaphoreType.DMA,
        pltpu.SemaphoreType.DMA,
        pltpu.SemaphoreType.DMA,
        pltpu.SemaphoreType.DMA,
        pltpu.SemaphoreType.DMA,
        pltpu.SemaphoreType.DMA,
    ],
)(_sc_body)


def kernel(node_feat, edge_index, W):
    edge_index = edge_index.astype(jnp.int32)
    src = edge_index[0]
    dst = edge_index[1]
    w1 = W[:D]
    w2 = W[D:]
    P, Q = _project_nodes(node_feat, w1, w2)
    return _sc_edges(P, Q, src, dst)


# speedup vs baseline: 7.5346x; 1.1151x over previous
import functools

import jax
import jax.numpy as jnp
from jax import lax
from jax.experimental import pallas as pl
from jax.experimental.pallas import tpu as pltpu
from jax.experimental.pallas import tpu_sc as plsc

N_NODES = 10000
N_EDGES = 320000
D = 128

NW = 32
BPW = N_EDGES // NW
C = 400
NCHUNK = BPW // C


def _mm_body(x_ref, w1_ref, w2_ref, p_ref, q_ref):
    x = x_ref[...]
    p_ref[...] = jnp.dot(x, w1_ref[...], preferred_element_type=jnp.float32)
    q_ref[...] = jnp.dot(x, w2_ref[...], preferred_element_type=jnp.float32)


def _project_nodes(node_feat, w1, w2):
    blk = 1000
    grid = N_NODES // blk
    return pl.pallas_call(
        _mm_body,
        grid=(grid,),
        in_specs=[
            pl.BlockSpec((blk, D), lambda i: (i, 0)),
            pl.BlockSpec((D, D), lambda i: (0, 0)),
            pl.BlockSpec((D, D), lambda i: (0, 0)),
        ],
        out_specs=[
            pl.BlockSpec((blk, D), lambda i: (i, 0)),
            pl.BlockSpec((blk, D), lambda i: (i, 0)),
        ],
        out_shape=[
            jax.ShapeDtypeStruct((N_NODES, D), jnp.float32),
            jax.ShapeDtypeStruct((N_NODES, D), jnp.float32),
        ],
    )(node_feat, w1, w2)


def _sc_body(p_hbm, q_hbm, src_hbm, dst_hbm, out_hbm,
             idxs, idxd, buf0, buf1, sp0, sq0, sp1, sq1, so0, so1):
    wid = lax.axis_index("s") * 2 + lax.axis_index("c")
    base = wid * BPW

    pltpu.sync_copy(src_hbm.at[pl.ds(base, BPW)], idxs)
    pltpu.sync_copy(dst_hbm.at[pl.ds(base, BPW)], idxd)

    bufs = ((buf0, sp0, sq0, so0), (buf1, sp1, sq1, so1))

    def start_p(t, b):
        buf, sp, _, _ = bufs[b]
        return pltpu.async_copy(p_hbm.at[idxs.at[pl.ds(t * C, C)]], buf, sp)

    def start_q(t, b):
        buf, _, sq, _ = bufs[b]
        return pltpu.async_copy(q_hbm.at[idxd.at[pl.ds(t * C, C)]], buf, sq,
                                add=True)

    def wait_p(b):
        buf, sp, _, _ = bufs[b]
        pltpu.make_async_copy(p_hbm.at[idxs.at[pl.ds(0, C)]], buf, sp).wait()

    def wait_q(b):
        buf, _, sq, _ = bufs[b]
        pltpu.make_async_copy(q_hbm.at[idxd.at[pl.ds(0, C)]], buf, sq).wait()

    def start_out(t, b):
        buf, _, _, so = bufs[b]
        return pltpu.async_copy(buf, out_hbm.at[pl.ds(base + t * C, C)], so)

    def wait_out(b):
        buf, _, _, so = bufs[b]
        pltpu.make_async_copy(buf, out_hbm.at[pl.ds(base, C)], so).wait()

    def lrelu_rows(buf, lo, hi):
        @plsc.parallel_loop(lo, hi, step=1, unroll=2)
        def _(r):
            for j in range(D // 16):
                sl = pl.ds(j * 16, 16)
                v = buf[r, sl]
                buf[r, sl] = jnp.maximum(v, v * 0.01)

    start_p(0, 0)
    wait_p(0)
    start_q(0, 0)

    def half_iter(t, b):
        buf = bufs[b][0]
        wait_q(b)

        @pl.when(t + 1 < NCHUNK)
        def _():
            @pl.when(t >= 1)
            def _():
                wait_out(1 - b)
            start_p(t + 1, 1 - b)

        lrelu_rows(buf, 0, C // 2)

        @pl.when(t + 1 < NCHUNK)
        def _():
            wait_p(1 - b)
            start_q(t + 1, 1 - b)

        lrelu_rows(buf, C // 2, C)
        start_out(t, b)

    def pair(g, carry):
        half_iter(2 * g, 0)
        half_iter(2 * g + 1, 1)
        return carry

    lax.fori_loop(0, (NCHUNK - 1) // 2, pair, 0)
    half_iter(NCHUNK - 1, 0)
    wait_out(1)
    wait_out(0)


_sc_edges = functools.partial(
    pl.kernel,
    out_type=jax.ShapeDtypeStruct((N_EDGES, D), jnp.float32),
    mesh=plsc.VectorSubcoreMesh(core_axis_name="c", subcore_axis_name="s"),
    scratch_types=[
        pltpu.VMEM((BPW,), jnp.int32),
        pltpu.VMEM((BPW,), jnp.int32),
        pltpu.VMEM((C, D), jnp.float32),
        pltpu.VMEM((C, D), jnp.float32),
        pltpu.SemaphoreType.DMA,
        pltpu.SemaphoreType.DMA,
        pltpu.SemaphoreType.DMA,
        pltpu.SemaphoreType.DMA,
        pltpu.SemaphoreType.DMA,
        pltpu.SemaphoreType.DMA,
    ],
)(_sc_body)


def kernel(node_feat, edge_index, W):
    edge_index = edge_index.astype(jnp.int32)
    src = edge_index[0]
    dst = edge_index[1]
    w1 = W[:D]
    w2 = W[D:]
    P, Q = _project_nodes(node_feat, w1, w2)
    return _sc_edges(P, Q, src, dst)


# speedup vs baseline: 7.7761x; 1.0321x over previous
import functools

import jax
import jax.numpy as jnp
from jax import lax
from jax.experimental import pallas as pl
from jax.experimental.pallas import tpu as pltpu
from jax.experimental.pallas import tpu_sc as plsc

N_NODES = 10000
N_EDGES = 320000
D = 128

NW = 32
BPW = N_EDGES // NW
C = 200
NCHUNK = BPW // C


def _mm_body(x_ref, w1_ref, w2_ref, p_ref, q_ref):
    x = x_ref[...]
    p_ref[...] = jnp.dot(x, w1_ref[...], preferred_element_type=jnp.float32)
    q_ref[...] = jnp.dot(x, w2_ref[...], preferred_element_type=jnp.float32)


def _project_nodes(node_feat, w1, w2):
    blk = 1000
    grid = N_NODES // blk
    return pl.pallas_call(
        _mm_body,
        grid=(grid,),
        in_specs=[
            pl.BlockSpec((blk, D), lambda i: (i, 0)),
            pl.BlockSpec((D, D), lambda i: (0, 0)),
            pl.BlockSpec((D, D), lambda i: (0, 0)),
        ],
        out_specs=[
            pl.BlockSpec((blk, D), lambda i: (i, 0)),
            pl.BlockSpec((blk, D), lambda i: (i, 0)),
        ],
        out_shape=[
            jax.ShapeDtypeStruct((N_NODES, D), jnp.float32),
            jax.ShapeDtypeStruct((N_NODES, D), jnp.float32),
        ],
    )(node_feat, w1, w2)


def _sc_body(p_hbm, q_hbm, src_hbm, dst_hbm, out_hbm,
             idxs, idxd, buf0, buf1, buf2,
             sp0, sq0, so0, sp1, sq1, so1, sp2, sq2, so2):
    wid = lax.axis_index("s") * 2 + lax.axis_index("c")
    base = wid * BPW

    pltpu.sync_copy(src_hbm.at[pl.ds(base, BPW)], idxs)
    pltpu.sync_copy(dst_hbm.at[pl.ds(base, BPW)], idxd)

    bufs = ((buf0, sp0, sq0, so0),
            (buf1, sp1, sq1, so1),
            (buf2, sp2, sq2, so2))

    def start_p(t, b):
        buf, sp, _, _ = bufs[b]
        return pltpu.async_copy(p_hbm.at[idxs.at[pl.ds(t * C, C)]], buf, sp)

    def start_q(t, b):
        buf, _, sq, _ = bufs[b]
        return pltpu.async_copy(q_hbm.at[idxd.at[pl.ds(t * C, C)]], buf, sq,
                                add=True)

    def wait_p(b):
        buf, sp, _, _ = bufs[b]
        pltpu.make_async_copy(p_hbm.at[idxs.at[pl.ds(0, C)]], buf, sp).wait()

    def wait_q(b):
        buf, _, sq, _ = bufs[b]
        pltpu.make_async_copy(q_hbm.at[idxd.at[pl.ds(0, C)]], buf, sq).wait()

    def start_out(t, b):
        buf, _, _, so = bufs[b]
        return pltpu.async_copy(buf, out_hbm.at[pl.ds(base + t * C, C)], so)

    def wait_out(b):
        buf, _, _, so = bufs[b]
        pltpu.make_async_copy(buf, out_hbm.at[pl.ds(base, C)], so).wait()

    def lrelu_rows(buf, lo, hi):
        @plsc.parallel_loop(lo, hi, step=1, unroll=2)
        def _(r):
            for j in range(D // 16):
                sl = pl.ds(j * 16, 16)
                v = buf[r, sl]
                buf[r, sl] = jnp.maximum(v, v * 0.01)

    start_p(0, 0)
    wait_p(0)
    start_q(0, 0)
    start_p(1, 1)

    def step(t, b):
        buf = bufs[b][0]
        b1 = (b + 1) % 3
        b2 = (b + 2) % 3
        wait_q(b)
        lrelu_rows(buf, 0, C // 2)

        @pl.when(t + 1 < NCHUNK)
        def _():
            wait_p(b1)
            start_q(t + 1, b1)

        lrelu_rows(buf, C // 2, C)

        @pl.when(t + 2 < NCHUNK)
        def _():
            @pl.when(t >= 1)
            def _():
                wait_out(b2)
            start_p(t + 2, b2)

        start_out(t, b)

    def group(g, carry):
        step(3 * g, 0)
        step(3 * g + 1, 1)
        step(3 * g + 2, 2)
        return carry

    lax.fori_loop(0, (NCHUNK - 2) // 3, group, 0)
    step(NCHUNK - 2, 0)
    step(NCHUNK - 1, 1)
    wait_out(2)
    wait_out(0)
    wait_out(1)


_sc_edges = functools.partial(
    pl.kernel,
    out_type=jax.ShapeDtypeStruct((N_EDGES, D), jnp.float32),
    mesh=plsc.VectorSubcoreMesh(core_axis_name="c", subcore_axis_name="s"),
    scratch_types=[
        pltpu.VMEM((BPW,), jnp.int32),
        pltpu.VMEM((BPW,), jnp.int32),
        pltpu.VMEM((C, D), jnp.float32),
        pltpu.VMEM((C, D), jnp.float32),
        pltpu.VMEM((C, D), jnp.float32),
        pltpu.SemaphoreType.DMA,
        pltpu.SemaphoreType.DMA,
        pltpu.SemaphoreType.DMA,
        pltpu.SemaphoreType.DMA,
        pltpu.SemaphoreType.DMA,
        pltpu.SemaphoreType.DMA,
        pltpu.SemaphoreType.DMA,
        pltpu.SemaphoreType.DMA,
        pltpu.SemaphoreType.DMA,
    ],
)(_sc_body)


def kernel(node_feat, edge_index, W):
    edge_index = edge_index.astype(jnp.int32)
    src = edge_index[0]
    dst = edge_index[1]
    w1 = W[:D]
    w2 = W[D:]
    P, Q = _project_nodes(node_feat, w1, w2)
    return _sc_edges(P, Q, src, dst)


# speedup vs baseline: 7.9757x; 1.0257x over previous
import functools

import jax
import jax.numpy as jnp
from jax import lax
from jax.experimental import pallas as pl
from jax.experimental.pallas import tpu as pltpu
from jax.experimental.pallas import tpu_sc as plsc

N_NODES = 10000
N_EDGES = 320000
D = 128

NW = 32
BPW = N_EDGES // NW
C = 200
NCHUNK = BPW // C


def _mm_body(x_ref, w_ref, p_ref, q_ref):
    x = x_ref[...]
    p_ref[...] = jnp.dot(x, w_ref[:D], preferred_element_type=jnp.float32)
    q_ref[...] = jnp.dot(x, w_ref[D:], preferred_element_type=jnp.float32)


def _project_nodes(node_feat, w):
    return pl.pallas_call(
        _mm_body,
        out_shape=[
            jax.ShapeDtypeStruct((N_NODES, D), jnp.float32),
            jax.ShapeDtypeStruct((N_NODES, D), jnp.float32),
        ],
    )(node_feat, w)


def _sc_body(p_hbm, q_hbm, src_hbm, dst_hbm, out_hbm,
             idxs, idxd, buf0, buf1, buf2,
             sp0, sq0, so0, sp1, sq1, so1, sp2, sq2, so2):
    wid = lax.axis_index("s") * 2 + lax.axis_index("c")
    base = wid * BPW

    pltpu.sync_copy(src_hbm.at[pl.ds(base, BPW)], idxs)
    pltpu.sync_copy(dst_hbm.at[pl.ds(base, BPW)], idxd)

    bufs = ((buf0, sp0, sq0, so0),
            (buf1, sp1, sq1, so1),
            (buf2, sp2, sq2, so2))

    def start_p(t, b):
        buf, sp, _, _ = bufs[b]
        return pltpu.async_copy(p_hbm.at[idxs.at[pl.ds(t * C, C)]], buf, sp)

    def start_q(t, b):
        buf, _, sq, _ = bufs[b]
        return pltpu.async_copy(q_hbm.at[idxd.at[pl.ds(t * C, C)]], buf, sq,
                                add=True)

    def wait_p(b):
        buf, sp, _, _ = bufs[b]
        pltpu.make_async_copy(p_hbm.at[idxs.at[pl.ds(0, C)]], buf, sp).wait()

    def wait_q(b):
        buf, _, sq, _ = bufs[b]
        pltpu.make_async_copy(q_hbm.at[idxd.at[pl.ds(0, C)]], buf, sq).wait()

    def start_out(t, b):
        buf, _, _, so = bufs[b]
        return pltpu.async_copy(buf, out_hbm.at[pl.ds(base + t * C, C)], so)

    def wait_out(b):
        buf, _, _, so = bufs[b]
        pltpu.make_async_copy(buf, out_hbm.at[pl.ds(base, C)], so).wait()

    def lrelu_rows(buf, lo, hi):
        @plsc.parallel_loop(lo, hi, step=1, unroll=2)
        def _(r):
            for j in range(D // 16):
                sl = pl.ds(j * 16, 16)
                v = buf[r, sl]
                buf[r, sl] = jnp.maximum(v, v * 0.01)

    start_p(0, 0)
    wait_p(0)
    start_q(0, 0)
    start_p(1, 1)

    def step(t, b):
        buf = bufs[b][0]
        b1 = (b + 1) % 3
        b2 = (b + 2) % 3
        wait_q(b)
        lrelu_rows(buf, 0, C // 2)

        @pl.when(t + 1 < NCHUNK)
        def _():
            wait_p(b1)
            start_q(t + 1, b1)

        lrelu_rows(buf, C // 2, C)

        @pl.when(t + 2 < NCHUNK)
        def _():
            @pl.when(t >= 1)
            def _():
                wait_out(b2)
            start_p(t + 2, b2)

        start_out(t, b)

    def group(g, carry):
        step(3 * g, 0)
        step(3 * g + 1, 1)
        step(3 * g + 2, 2)
        return carry

    lax.fori_loop(0, (NCHUNK - 2) // 3, group, 0)
    step(NCHUNK - 2, 0)
    step(NCHUNK - 1, 1)
    wait_out(2)
    wait_out(0)
    wait_out(1)


_sc_edges = functools.partial(
    pl.kernel,
    out_type=jax.ShapeDtypeStruct((N_EDGES, D), jnp.float32),
    mesh=plsc.VectorSubcoreMesh(core_axis_name="c", subcore_axis_name="s"),
    scratch_types=[
        pltpu.VMEM((BPW,), jnp.int32),
        pltpu.VMEM((BPW,), jnp.int32),
        pltpu.VMEM((C, D), jnp.float32),
        pltpu.VMEM((C, D), jnp.float32),
        pltpu.VMEM((C, D), jnp.float32),
        pltpu.SemaphoreType.DMA,
        pltpu.SemaphoreType.DMA,
        pltpu.SemaphoreType.DMA,
        pltpu.SemaphoreType.DMA,
        pltpu.SemaphoreType.DMA,
        pltpu.SemaphoreType.DMA,
        pltpu.SemaphoreType.DMA,
        pltpu.SemaphoreType.DMA,
        pltpu.SemaphoreType.DMA,
    ],
)(_sc_body)


def kernel(node_feat, edge_index, W):
    edge_index = edge_index.astype(jnp.int32)
    src = edge_index[0]
    dst = edge_index[1]
    P, Q = _project_nodes(node_feat, W)
    return _sc_edges(P, Q, src, dst)


# speedup vs baseline: 8.4959x; 1.0652x over previous
import functools

import jax
import jax.numpy as jnp
from jax import lax
from jax.experimental import pallas as pl
from jax.experimental.pallas import tpu as pltpu
from jax.experimental.pallas import tpu_sc as plsc

N_NODES = 10000
N_EDGES = 320000
D = 128

NW = 32
BPW = N_EDGES // NW
C = 200
NCHUNK = BPW // C


def _mm_body(x_ref, w_ref, ei_ref, p_ref, q_ref, src_ref, dst_ref):
    x = x_ref[...]
    p_ref[...] = jnp.dot(x, w_ref[:D], preferred_element_type=jnp.float32)
    q_ref[...] = jnp.dot(x, w_ref[D:], preferred_element_type=jnp.float32)
    src_ref[...] = ei_ref[0]
    dst_ref[...] = ei_ref[1]


def _project_nodes(node_feat, w, edge_index):
    return pl.pallas_call(
        _mm_body,
        out_shape=[
            jax.ShapeDtypeStruct((N_NODES, D), jnp.float32),
            jax.ShapeDtypeStruct((N_NODES, D), jnp.float32),
            jax.ShapeDtypeStruct((N_EDGES,), jnp.int32),
            jax.ShapeDtypeStruct((N_EDGES,), jnp.int32),
        ],
    )(node_feat, w, edge_index)


def _sc_body(p_hbm, q_hbm, src_hbm, dst_hbm, out_hbm,
             idxs, idxd, buf0, buf1, buf2,
             sp0, sq0, so0, sp1, sq1, so1, sp2, sq2, so2):
    wid = lax.axis_index("s") * 2 + lax.axis_index("c")
    base = wid * BPW

    pltpu.sync_copy(src_hbm.at[pl.ds(base, BPW)], idxs)
    pltpu.sync_copy(dst_hbm.at[pl.ds(base, BPW)], idxd)

    bufs = ((buf0, sp0, sq0, so0),
            (buf1, sp1, sq1, so1),
            (buf2, sp2, sq2, so2))

    def start_p(t, b):
        buf, sp, _, _ = bufs[b]
        return pltpu.async_copy(p_hbm.at[idxs.at[pl.ds(t * C, C)]], buf, sp)

    def start_q(t, b):
        buf, _, sq, _ = bufs[b]
        return pltpu.async_copy(q_hbm.at[idxd.at[pl.ds(t * C, C)]], buf, sq,
                                add=True)

    def wait_p(b):
        buf, sp, _, _ = bufs[b]
        pltpu.make_async_copy(p_hbm.at[idxs.at[pl.ds(0, C)]], buf, sp).wait()

    def wait_q(b):
        buf, _, sq, _ = bufs[b]
        pltpu.make_async_copy(q_hbm.at[idxd.at[pl.ds(0, C)]], buf, sq).wait()

    def start_out(t, b):
        buf, _, _, so = bufs[b]
        return pltpu.async_copy(buf, out_hbm.at[pl.ds(base + t * C, C)], so)

    def wait_out(b):
        buf, _, _, so = bufs[b]
        pltpu.make_async_copy(buf, out_hbm.at[pl.ds(base, C)], so).wait()

    def lrelu_rows(buf, lo, hi):
        @plsc.parallel_loop(lo, hi, step=1, unroll=2)
        def _(r):
            for j in range(D // 16):
                sl = pl.ds(j * 16, 16)
                v = buf[r, sl]
                buf[r, sl] = jnp.maximum(v, v * 0.01)

    start_p(0, 0)
    wait_p(0)
    start_q(0, 0)
    start_p(1, 1)

    def step(t, b):
        buf = bufs[b][0]
        b1 = (b + 1) % 3
        b2 = (b + 2) % 3
        wait_q(b)
        lrelu_rows(buf, 0, C // 2)

        @pl.when(t + 1 < NCHUNK)
        def _():
            wait_p(b1)
            start_q(t + 1, b1)

        lrelu_rows(buf, C // 2, C)

        @pl.when(t + 2 < NCHUNK)
        def _():
            @pl.when(t >= 1)
            def _():
                wait_out(b2)
            start_p(t + 2, b2)

        start_out(t, b)

    def group(g, carry):
        step(3 * g, 0)
        step(3 * g + 1, 1)
        step(3 * g + 2, 2)
        return carry

    lax.fori_loop(0, (NCHUNK - 2) // 3, group, 0)
    step(NCHUNK - 2, 0)
    step(NCHUNK - 1, 1)
    wait_out(2)
    wait_out(0)
    wait_out(1)


_sc_edges = functools.partial(
    pl.kernel,
    out_type=jax.ShapeDtypeStruct((N_EDGES, D), jnp.float32),
    mesh=plsc.VectorSubcoreMesh(core_axis_name="c", subcore_axis_name="s"),
    scratch_types=[
        pltpu.VMEM((BPW,), jnp.int32),
        pltpu.VMEM((BPW,), jnp.int32),
        pltpu.VMEM((C, D), jnp.float32),
        pltpu.VMEM((C, D), jnp.float32),
        pltpu.VMEM((C, D), jnp.float32),
        pltpu.SemaphoreType.DMA,
        pltpu.SemaphoreType.DMA,
        pltpu.SemaphoreType.DMA,
        pltpu.SemaphoreType.DMA,
        pltpu.SemaphoreType.DMA,
        pltpu.SemaphoreType.DMA,
        pltpu.SemaphoreType.DMA,
        pltpu.SemaphoreType.DMA,
        pltpu.SemaphoreType.DMA,
    ],
)(_sc_body)


def kernel(node_feat, edge_index, W):
    edge_index = edge_index.astype(jnp.int32)
    P, Q, src, dst = _project_nodes(node_feat, W, edge_index)
    return _sc_edges(P, Q, src, dst)
